# Initial kernel scaffold; baseline (speedup 1.0000x reference)
import jax
import jax.numpy as jnp
from jax.experimental import pallas as pl


def kernel(drug_indices, cell_indices, drug_features, cell_features, drug_edge_index, cell_edge_index, Wd1, bd1, Wc1, bc1, Wc2, bc2, Wg_d, as_d, ad_d, bg_d, Wg1, as1, ad1, bg1, Wg2, as2, ad2, bg2, Wr1, br1, Wr2, br2, Wr3, br3):
    raise NotImplementedError("write your pallas kernel here")


# speedup vs baseline: 1.0311x; 1.0311x over previous
import functools

import jax
import jax.numpy as jnp
from jax import lax
from jax.experimental import pallas as pl

N_DRUG = 10000
N_CELL = 10000


def _mm_body(x_ref, w_ref, bin_ref, bout_ref, a_ref, out_ref, es_ref,
             pre_relu, post_relu, emit_att):
    x = x_ref[...]
    if pre_relu:
        x = jnp.maximum(x + bin_ref[...], 0.0)
    h = jnp.dot(x, w_ref[...], preferred_element_type=jnp.float32)
    if emit_att:
        es_ref[...] = jnp.dot(h, a_ref[...], preferred_element_type=jnp.float32)
    h = h + bout_ref[...]
    if post_relu:
        h = jnp.maximum(h, 0.0)
    out_ref[...] = h


def _mm(x, w, b_out, *, b_in=None, pre_relu=False, post_relu=False,
        a_mat=None, block_m=1000):
    m, k = x.shape
    n = w.shape[1]
    emit_att = a_mat is not None
    grid = (m // block_m,)
    if b_in is None:
        b_in = jnp.zeros((1, k), jnp.float32)
    if a_mat is None:
        a_mat = jnp.zeros((n, 128), jnp.float32)
    in_specs = [
        pl.BlockSpec((block_m, k), lambda i: (i, 0)),
        pl.BlockSpec((k, n), lambda i: (0, 0)),
        pl.BlockSpec((1, k), lambda i: (0, 0)),
        pl.BlockSpec((1, n), lambda i: (0, 0)),
        pl.BlockSpec((n, 128), lambda i: (0, 0)),
    ]
    out_shape = [jax.ShapeDtypeStruct((m, n), jnp.float32),
                 jax.ShapeDtypeStruct((m, 128), jnp.float32)]
    out_specs = [pl.BlockSpec((block_m, n), lambda i: (i, 0)),
                 pl.BlockSpec((block_m, 128), lambda i: (i, 0))]
    body = functools.partial(_mm_body, pre_relu=pre_relu, post_relu=post_relu,
                             emit_att=emit_att)
    out, es = pl.pallas_call(
        body, grid=grid, in_specs=in_specs, out_specs=out_specs,
        out_shape=out_shape)(x, w, b_in.reshape(1, k), b_out.reshape(1, n),
                             a_mat)
    if emit_att:
        return out, es[:, 0], es[:, 1]
    return out


def _head_body(x_ref, bcat_ref, w1_ref, b1_ref, w2_ref, b2_ref, w3_ref,
               b3_ref, out_ref):
    x = jnp.maximum(x_ref[...] + bcat_ref[...], 0.0)
    h = jnp.dot(x, w1_ref[...], preferred_element_type=jnp.float32) + b1_ref[...]
    h = jnp.where(h > 0, h, jnp.exp(jnp.minimum(h, 0.0)) - 1.0)
    h = jnp.dot(h, w2_ref[...], preferred_element_type=jnp.float32) + b2_ref[...]
    h = jnp.where(h > 0, h, jnp.exp(jnp.minimum(h, 0.0)) - 1.0)
    out_ref[...] = jnp.dot(h, w3_ref[...], preferred_element_type=jnp.float32) + b3_ref[...]


def _head(comb_raw, bcat, w1, b1, w2, b2, w3, b3, block_m=1024):
    m, k = comb_raw.shape
    w3p = jnp.pad(w3, ((0, 0), (0, 127)))
    b3p = jnp.pad(b3, (0, 127))
    grid = (m // block_m,)
    in_specs = [
        pl.BlockSpec((block_m, k), lambda i: (i, 0)),
        pl.BlockSpec((1, k), lambda i: (0, 0)),
        pl.BlockSpec((512, 512), lambda i: (0, 0)),
        pl.BlockSpec((1, 512), lambda i: (0, 0)),
        pl.BlockSpec((512, 512), lambda i: (0, 0)),
        pl.BlockSpec((1, 512), lambda i: (0, 0)),
        pl.BlockSpec((512, 128), lambda i: (0, 0)),
        pl.BlockSpec((1, 128), lambda i: (0, 0)),
    ]
    out = pl.pallas_call(
        _head_body, grid=grid, in_specs=in_specs,
        out_specs=pl.BlockSpec((block_m, 128), lambda i: (i, 0)),
        out_shape=jax.ShapeDtypeStruct((m, 128), jnp.float32))(
            comb_raw, bcat.reshape(1, k), w1, b1.reshape(1, 512), w2,
            b2.reshape(1, 512), w3p, b3p.reshape(1, 128))
    return out[:, :1]


def _gat_edges(h, es, ed, edge_index, n):
    src = edge_index[0]
    dst = edge_index[1]
    e = jax.nn.leaky_relu(es[src] + ed[dst], negative_slope=0.2)
    m = jax.ops.segment_max(e, dst, num_segments=n)
    m = jnp.where(jnp.isfinite(m), m, 0.0)
    ex = jnp.exp(e - m[dst])
    s = jax.ops.segment_sum(ex, dst, num_segments=n)
    alpha = ex / (s[dst] + 1e-16)
    return jax.ops.segment_sum(h[src] * alpha[:, None], dst, num_segments=n)


def kernel(drug_indices, cell_indices, drug_features, cell_features,
           drug_edge_index, cell_edge_index, Wd1, bd1, Wc1, bc1, Wc2, bc2,
           Wg_d, as_d, ad_d, bg_d, Wg1, as1, ad1, bg1, Wg2, as2, ad2, bg2,
           Wr1, br1, Wr2, br2, Wr3, br3):
    def att_mat(a_s, a_d):
        return jnp.pad(jnp.stack([a_s, a_d], axis=1), ((0, 0), (0, 126)))

    d0 = _mm(drug_features, Wd1, bd1, post_relu=True)
    hd, es_d, ed_d = _mm(d0, Wg_d, jnp.zeros((256,), jnp.float32),
                         a_mat=att_mat(as_d, ad_d))
    gd_raw = _gat_edges(hd, es_d, ed_d, drug_edge_index, N_DRUG)

    c0 = _mm(cell_features, Wc1, bc1, post_relu=True)
    c1 = _mm(c0, Wc2, bc2, post_relu=True)
    h1, es1, ed1 = _mm(c1, Wg1, jnp.zeros((1024,), jnp.float32),
                       a_mat=att_mat(as1, ad1))
    g1_raw = _gat_edges(h1, es1, ed1, cell_edge_index, N_CELL)
    h2, es2, ed2 = _mm(g1_raw, Wg2, jnp.zeros((256,), jnp.float32),
                       b_in=bg1, pre_relu=True, a_mat=att_mat(as2, ad2))
    g2_raw = _gat_edges(h2, es2, ed2, cell_edge_index, N_CELL)

    comb_raw = jnp.concatenate(
        [gd_raw[drug_indices], g2_raw[cell_indices]], axis=-1)
    bcat = jnp.concatenate([bg_d, bg2])
    return _head(comb_raw, bcat, Wr1, br1, Wr2, br2, Wr3, br3)


# speedup vs baseline: 4.5147x; 4.3785x over previous
import functools

import jax
import jax.numpy as jnp
from jax import lax
from jax.experimental import pallas as pl
from jax.experimental.pallas import tpu as pltpu
from jax.experimental.pallas import tpu_sc as plsc

N_DRUG = 10000
N_CELL = 10000
N_PAD = 10240
ROWS_T = N_PAD // 16
CW = 128
HALF = 5056
HROWS = 5120
RT_H = HROWS // 16
DUMMY = 5056


def _mm_body(x_ref, w_ref, bin_ref, bout_ref, a_ref, out_ref, es_ref,
             pre_relu, post_relu, emit_att):
    x = x_ref[...]
    if pre_relu:
        x = jnp.maximum(x + bin_ref[...], 0.0)
    h = jnp.dot(x, w_ref[...], preferred_element_type=jnp.float32)
    if emit_att:
        es = jnp.sum(h * a_ref[0, :][None, :], axis=1, keepdims=True)
        ed = jnp.sum(h * a_ref[1, :][None, :], axis=1, keepdims=True)
        es_ref[...] = jnp.concatenate(
            [es, ed, jnp.zeros((h.shape[0], 126), jnp.float32)], axis=1)
    h = h + bout_ref[...]
    if post_relu:
        h = jnp.maximum(h, 0.0)
    out_ref[...] = h


def _mm(x, w, b_out, *, b_in=None, pre_relu=False, post_relu=False,
        a_mat=None, block_m=1000):
    m, k = x.shape
    n = w.shape[1]
    emit_att = a_mat is not None
    grid = (m // block_m,)
    if b_in is None:
        b_in = jnp.zeros((1, k), jnp.float32)
    if a_mat is None:
        a_mat = jnp.zeros((2, n), jnp.float32)
    in_specs = [
        pl.BlockSpec((block_m, k), lambda i: (i, 0)),
        pl.BlockSpec((k, n), lambda i: (0, 0)),
        pl.BlockSpec((1, k), lambda i: (0, 0)),
        pl.BlockSpec((1, n), lambda i: (0, 0)),
        pl.BlockSpec((2, n), lambda i: (0, 0)),
    ]
    out_shape = [jax.ShapeDtypeStruct((m, n), jnp.float32),
                 jax.ShapeDtypeStruct((m, 128), jnp.float32)]
    out_specs = [pl.BlockSpec((block_m, n), lambda i: (i, 0)),
                 pl.BlockSpec((block_m, 128), lambda i: (i, 0))]
    body = functools.partial(_mm_body, pre_relu=pre_relu, post_relu=post_relu,
                             emit_att=emit_att)
    out, es = pl.pallas_call(
        body, grid=grid, in_specs=in_specs, out_specs=out_specs,
        out_shape=out_shape)(x, w, b_in.reshape(1, k), b_out.reshape(1, n),
                             a_mat)
    if emit_att:
        return out, es[:, 0], es[:, 1]
    return out


def _head_body(x_ref, bcat_ref, w1_ref, b1_ref, w2_ref, b2_ref, w3_ref,
               b3_ref, out_ref):
    x = jnp.maximum(x_ref[...] + bcat_ref[...], 0.0)
    h = jnp.dot(x, w1_ref[...], preferred_element_type=jnp.float32) + b1_ref[...]
    h = jnp.where(h > 0, h, jnp.exp(jnp.minimum(h, 0.0)) - 1.0)
    h = jnp.dot(h, w2_ref[...], preferred_element_type=jnp.float32) + b2_ref[...]
    h = jnp.where(h > 0, h, jnp.exp(jnp.minimum(h, 0.0)) - 1.0)
    out_ref[...] = jnp.dot(h, w3_ref[...], preferred_element_type=jnp.float32) + b3_ref[...]


def _head(comb_raw, bcat, w1, b1, w2, b2, w3, b3, block_m=1024):
    m, k = comb_raw.shape
    w3p = jnp.pad(w3, ((0, 0), (0, 127)))
    b3p = jnp.pad(b3, (0, 127))
    grid = (m // block_m,)
    in_specs = [
        pl.BlockSpec((block_m, k), lambda i: (i, 0)),
        pl.BlockSpec((1, k), lambda i: (0, 0)),
        pl.BlockSpec((512, 512), lambda i: (0, 0)),
        pl.BlockSpec((1, 512), lambda i: (0, 0)),
        pl.BlockSpec((512, 512), lambda i: (0, 0)),
        pl.BlockSpec((1, 512), lambda i: (0, 0)),
        pl.BlockSpec((512, 128), lambda i: (0, 0)),
        pl.BlockSpec((1, 128), lambda i: (0, 0)),
    ]
    out = pl.pallas_call(
        _head_body, grid=grid, in_specs=in_specs,
        out_specs=pl.BlockSpec((block_m, 128), lambda i: (i, 0)),
        out_shape=jax.ShapeDtypeStruct((m, 128), jnp.float32))(
            comb_raw, bcat.reshape(1, k), w1, b1.reshape(1, 512), w2,
            b2.reshape(1, 512), w3p, b3p.reshape(1, 128))
    return out[:, :1]


def _gat_edge_kernel(nch, n_chunks):
    def body(pe_hbm, es_hbm, ed_hbm, hstack_hbm, out_hbm,
             ebuf, dstb, es_t, ed_t, s_t, alpha_t, exv, idxb, dstl, msg,
             zbuf, zs, acc_s, acc, sem, semE, semS, semG2, semS2):
        c = lax.axis_index("c")
        s_ax = lax.axis_index("s")
        pbase = s_ax * (nch * 256)

        def pref(jn, slot):
            pltpu.async_copy(pe_hbm.at[pl.ds(pbase + jn * 256, 256)],
                             ebuf.at[slot], semE)

        def wait_pref():
            pltpu.make_async_copy(pe_hbm.at[pl.ds(pbase, 256)],
                                  ebuf.at[0], semE).wait()

        def wait_scatter_a():
            pltpu.make_async_copy(exv.at[0], acc_s.at[pl.ds(0, 128)],
                                  semS).wait()

        pref(0, 0)
        def zrow(r, _):
            for v in range(8):
                zbuf[r, pl.ds(v * 16, 16)] = jnp.zeros((16,), jnp.float32)
            return 0
        lax.fori_loop(0, 32, zrow, 0)
        def zsrow(i, _):
            zs[pl.ds(i * 16, 16)] = jnp.zeros((16,), jnp.float32)
            return 0
        lax.fori_loop(0, 40, zsrow, 0)
        pltpu.sync_copy(zs.at[pl.ds(0, ROWS_T)],
                        acc_s.at[pl.ds(s_ax * ROWS_T, ROWS_T)])
        nz = RT_H // 32 + (1 if RT_H % 32 else 0)
        for p in range(nz):
            off = p * 32
            ln = min(32, RT_H - off)
            pltpu.sync_copy(zbuf.at[pl.ds(0, ln)],
                            acc.at[pl.ds(s_ax * RT_H + off, ln)])
        pltpu.sync_copy(es_hbm, es_t)
        pltpu.sync_copy(ed_hbm, ed_t)
        plsc.subcore_barrier()

        def a_body(j, _):
            jm = lax.rem(j, 2)
            wait_pref()
            pref(lax.min(j + 1, nch - 1), lax.rem(j + 1, 2))

            @pl.when(j >= 1)
            def _():
                wait_scatter_a()
            for v in range(8):
                sl = pl.ds(v * 16, 16)
                sv = ebuf[jm, sl]
                dv = ebuf[jm, pl.ds(128 + v * 16, 16)]
                dstb[jm, sl] = dv
                e = (plsc.load_gather(es_t, [sv])
                     + plsc.load_gather(ed_t, [dv]))
                e = jnp.where(e >= 0.0, e, 0.2 * e)
                exv[jm, sl] = jnp.exp(e)
            pltpu.async_copy(exv.at[jm], acc_s.at[dstb.at[jm]], semS,
                             add=True)
            return 0
        lax.fori_loop(0, nch, a_body, 0)
        wait_pref()
        wait_scatter_a()
        plsc.subcore_barrier()
        pltpu.sync_copy(acc_s, s_t)

        for cid in range(n_chunks):
            plsc.subcore_barrier()
            pref(0, 0)

            def b_body(j, _):
                jm = lax.rem(j, 2)
                wait_pref()
                pref(lax.min(j + 1, nch - 1), lax.rem(j + 1, 2))
                for v in range(8):
                    sl = pl.ds(v * 16, 16)
                    idxb[v // 4, pl.ds((v % 4) * 16, 16)] = (
                        ebuf[jm, sl] + cid * 10000)
                @pl.when(j >= 1)
                def _():
                    pltpu.make_async_copy(
                        msg.at[0], acc.at[pl.ds(0, 64)], semS2).wait()
                    pltpu.make_async_copy(
                        msg.at[0], acc.at[pl.ds(0, 64)], semS2).wait()
                gh0 = pltpu.async_copy(hstack_hbm.at[idxb.at[0]],
                                       msg.at[0], sem)
                gh1 = pltpu.async_copy(hstack_hbm.at[idxb.at[1]],
                                       msg.at[1], semG2)
                for v in range(8):
                    dl = ebuf[jm, pl.ds(128 + v * 16, 16)] - c * HALF
                    bad = (dl < 0) | (dl >= HALF)
                    dstl[v // 4, pl.ds((v % 4) * 16, 16)] = (
                        jnp.where(bad, DUMMY, dl))
                if cid == 0:
                    for v in range(8):
                        sl = pl.ds(v * 16, 16)
                        sv = ebuf[jm, sl]
                        dv = ebuf[jm, pl.ds(128 + v * 16, 16)]
                        e = (plsc.load_gather(es_t, [sv])
                             + plsc.load_gather(ed_t, [dv]))
                        e = jnp.where(e >= 0.0, e, 0.2 * e)
                        sg = plsc.load_gather(s_t, [dv])
                        alpha_t[pl.ds(j * 128 + v * 16, 16)] = (
                            jnp.exp(e) / (sg + 1e-16))
                for sub, gh in ((0, gh0), (1, gh1)):
                    gh.wait()
                    def w_body(g, _):
                        for i in range(16):
                            av = plsc.load_gather(
                                alpha_t,
                                [jnp.full((16,),
                                          j * 128 + sub * 64 + g * 16 + i,
                                          jnp.int32)])
                            e2 = g * 16 + i
                            for v in range(8):
                                sl = pl.ds(v * 16, 16)
                                msg[sub, e2, sl] = msg[sub, e2, sl] * av
                        return 0
                    lax.fori_loop(0, 4, w_body, 0)
                    pltpu.async_copy(msg.at[sub], acc.at[dstl.at[sub]],
                                     semS2, add=True)
                return 0
            lax.fori_loop(0, nch, b_body, 0)
            wait_pref()
            pltpu.make_async_copy(msg.at[0], acc.at[pl.ds(0, 64)],
                                  semS2).wait()
            pltpu.make_async_copy(msg.at[0], acc.at[pl.ds(0, 64)],
                                  semS2).wait()
            plsc.subcore_barrier()

            for p in range(5):
                off = p * 64
                ln = min(64, RT_H - off)
                pltpu.sync_copy(acc.at[pl.ds(s_ax * RT_H + off, ln)],
                                msg.at[0])
                pltpu.sync_copy(
                    msg.at[0],
                    out_hbm.at[pl.ds(cid * 2 * HROWS + c * HROWS
                                     + s_ax * RT_H + off, ln)])
            if cid + 1 < n_chunks:
                for p in range(nz):
                    off = p * 32
                    ln = min(32, RT_H - off)
                    pltpu.sync_copy(zbuf.at[pl.ds(0, ln)],
                                    acc.at[pl.ds(s_ax * RT_H + off, ln)])
    return body


def _gat_edges(h, es, ed, edge_index, n):
    dout = h.shape[1]
    n_chunks = dout // CW
    src = edge_index[0]
    dst = edge_index[1]
    e_edges = src.shape[0]
    e_pad = ((e_edges + 2047) // 2048) * 2048
    nch = e_pad // 2048
    src_p = jnp.pad(src, (0, e_pad - e_edges)).astype(jnp.int32)
    dst_p = jnp.pad(dst, (0, e_pad - e_edges),
                    constant_values=n).astype(jnp.int32)
    pe = jnp.concatenate(
        [src_p.reshape(16, nch, 128), dst_p.reshape(16, nch, 128)],
        axis=2).reshape(-1)
    es_p = jnp.pad(es, (0, N_PAD - n))
    ed_p = jnp.pad(ed, (0, N_PAD - n))
    hstack = jnp.concatenate(
        [h[:, cc * CW:(cc + 1) * CW] for cc in range(n_chunks)], axis=0)

    mesh = plsc.VectorSubcoreMesh(core_axis_name="c", subcore_axis_name="s")
    e16 = nch * 128
    k = pl.kernel(
        _gat_edge_kernel(nch, n_chunks),
        mesh=mesh,
        compiler_params=pltpu.CompilerParams(needs_layout_passes=False),
        out_type=jax.ShapeDtypeStruct((n_chunks * 2 * HROWS, CW), jnp.float32),
        scratch_types=[
            pltpu.VMEM((2, 256), jnp.int32),
            pltpu.VMEM((2, 128), jnp.int32),
            pltpu.VMEM((N_PAD,), jnp.float32),
            pltpu.VMEM((N_PAD,), jnp.float32),
            pltpu.VMEM((N_PAD,), jnp.float32),
            pltpu.VMEM((e16,), jnp.float32),
            pltpu.VMEM((2, 128), jnp.float32),
            pltpu.VMEM((2, 64), jnp.int32),
            pltpu.VMEM((2, 64), jnp.int32),
            pltpu.VMEM((2, 64, CW), jnp.float32),
            pltpu.VMEM((32, CW), jnp.float32),
            pltpu.VMEM((640,), jnp.float32),
            pltpu.VMEM_SHARED((N_PAD,), jnp.float32),
            pltpu.VMEM_SHARED((HROWS, CW), jnp.float32),
            pltpu.SemaphoreType.DMA,
            pltpu.SemaphoreType.DMA,
            pltpu.SemaphoreType.DMA,
            pltpu.SemaphoreType.DMA,
            pltpu.SemaphoreType.DMA,
        ],
    )
    out_flat = k(pe, es_p, ed_p, hstack)
    out3 = out_flat.reshape(n_chunks, 2 * HROWS, CW)
    full = jnp.concatenate(
        [out3[:, :HALF], out3[:, HROWS:HROWS + (n - HALF)]], axis=1)
    return full.transpose(1, 0, 2).reshape(n, dout)


def _pair_gather_kernel(didx_hbm, cidx_hbm, gd_hbm, gc_hbm, od_hbm, oc_hbm,
                        idx_v, rows_v, sem):
    wid = lax.axis_index("s") * 2 + lax.axis_index("c")
    base = wid * 128
    pltpu.sync_copy(didx_hbm.at[pl.ds(base, 128)], idx_v)
    pltpu.async_copy(gd_hbm.at[idx_v], rows_v, sem).wait()
    pltpu.sync_copy(rows_v, od_hbm.at[pl.ds(base, 128)])
    pltpu.sync_copy(cidx_hbm.at[pl.ds(base, 128)], idx_v)
    pltpu.async_copy(gc_hbm.at[idx_v], rows_v, sem).wait()
    pltpu.sync_copy(rows_v, oc_hbm.at[pl.ds(base, 128)])


def _pair_gather(didx, cidx, gd, gc):
    mesh = plsc.VectorSubcoreMesh(core_axis_name="c", subcore_axis_name="s")
    b = didx.shape[0]
    k = pl.kernel(
        _pair_gather_kernel,
        mesh=mesh,
        out_type=[jax.ShapeDtypeStruct((b, 256), jnp.float32),
                  jax.ShapeDtypeStruct((b, 256), jnp.float32)],
        scratch_types=[
            pltpu.VMEM((128,), jnp.int32),
            pltpu.VMEM((128, 256), jnp.float32),
            pltpu.SemaphoreType.DMA,
        ],
    )
    return k(didx.astype(jnp.int32), cidx.astype(jnp.int32), gd, gc)


def kernel(drug_indices, cell_indices, drug_features, cell_features,
           drug_edge_index, cell_edge_index, Wd1, bd1, Wc1, bc1, Wc2, bc2,
           Wg_d, as_d, ad_d, bg_d, Wg1, as1, ad1, bg1, Wg2, as2, ad2, bg2,
           Wr1, br1, Wr2, br2, Wr3, br3):
    def att_mat(a_s, a_d):
        return jnp.stack([a_s, a_d], axis=0)

    d0 = _mm(drug_features, Wd1, bd1, post_relu=True)
    hd, es_d, ed_d = _mm(d0, Wg_d, jnp.zeros((256,), jnp.float32),
                         a_mat=att_mat(as_d, ad_d))
    gd_raw = _gat_edges(hd, es_d, ed_d, drug_edge_index, N_DRUG)

    c0 = _mm(cell_features, Wc1, bc1, post_relu=True)
    c1 = _mm(c0, Wc2, bc2, post_relu=True)
    h1, es1, ed1 = _mm(c1, Wg1, jnp.zeros((1024,), jnp.float32),
                       a_mat=att_mat(as1, ad1))
    g1_raw = _gat_edges(h1, es1, ed1, cell_edge_index, N_CELL)
    h2, es2, ed2 = _mm(g1_raw, Wg2, jnp.zeros((256,), jnp.float32),
                       b_in=bg1, pre_relu=True, a_mat=att_mat(as2, ad2))
    g2_raw = _gat_edges(h2, es2, ed2, cell_edge_index, N_CELL)

    od, oc = _pair_gather(drug_indices, cell_indices, gd_raw, g2_raw)
    comb_raw = jnp.concatenate([od, oc], axis=-1)
    bcat = jnp.concatenate([bg_d, bg2])
    return _head(comb_raw, bcat, Wr1, br1, Wr2, br2, Wr3, br3)
